# baseline (speedup 1.0000x reference)
import functools
import math

import jax
import jax.numpy as jnp
from jax import lax
from jax.experimental import pallas as pl
from jax.experimental.pallas import tpu as pltpu
from jax.experimental.pallas import tpu_sc as plsc

D_MODEL = 128
BATCH = 4
SEQ_LEN = 2048
TOTAL = BATCH * SEQ_LEN

NUM_CORES = 2
NUM_SUBCORES = 16
NUM_WORKERS = NUM_CORES * NUM_SUBCORES
LANES = 16

B_PER_W = TOTAL // NUM_WORKERS
CHUNK = 32
N_CHUNKS = B_PER_W // CHUNK

SCALE = math.sqrt(float(D_MODEL))

_mesh = plsc.VectorSubcoreMesh(core_axis_name="c", subcore_axis_name="s")


@functools.partial(
    pl.kernel,
    mesh=_mesh,
    out_type=jax.ShapeDtypeStruct((TOTAL, D_MODEL), jnp.float32),
    scratch_types=[
        pltpu.VMEM((N_CHUNKS, CHUNK), jnp.int32),
        pltpu.VMEM((B_PER_W, D_MODEL), jnp.float32),
    ]
    + [pltpu.SemaphoreType.DMA] * (2 * N_CHUNKS),
)
def _emb_lookup(idx_hbm, table_hbm, out_hbm, idx_v, rows_v, *sems):
    g_sems = sems[:N_CHUNKS]
    w_sems = sems[N_CHUNKS:]
    wid = lax.axis_index("s") * NUM_CORES + lax.axis_index("c")
    base = wid * B_PER_W

    pltpu.sync_copy(idx_hbm.at[pl.ds(wid * N_CHUNKS, N_CHUNKS)], idx_v)

    gathers = [
        pltpu.async_copy(
            table_hbm.at[idx_v.at[j]],
            rows_v.at[pl.ds(j * CHUNK, CHUNK)],
            g_sems[j],
        )
        for j in range(N_CHUNKS)
    ]

    writes = []
    for j in range(N_CHUNKS):
        gathers[j].wait()

        def scale_rows(r, carry, j=j):
            row = j * CHUNK + r
            for c in range(D_MODEL // LANES):
                sl = pl.ds(c * LANES, LANES)
                rows_v[row, sl] = rows_v[row, sl] * SCALE
            return carry

        lax.fori_loop(0, CHUNK, scale_rows, 0, unroll=4)

        writes.append(
            pltpu.async_copy(
                rows_v.at[pl.ds(j * CHUNK, CHUNK)],
                out_hbm.at[pl.ds(base + j * CHUNK, CHUNK)],
                w_sems[j],
            )
        )
    for w in writes:
        w.wait()


def kernel(indices, table):
    idx = indices.astype(jnp.int32).reshape(NUM_WORKERS * N_CHUNKS, CHUNK)
    out = _emb_lookup(idx, table)
    return out.reshape(indices.shape + (D_MODEL,))


# speedup vs baseline: 1.0258x; 1.0258x over previous
import functools
import math

import jax
import jax.numpy as jnp
from jax import lax
from jax.experimental import pallas as pl
from jax.experimental.pallas import tpu as pltpu
from jax.experimental.pallas import tpu_sc as plsc

D_MODEL = 128
BATCH = 4
SEQ_LEN = 2048
TOTAL = BATCH * SEQ_LEN

NUM_CORES = 2
NUM_SUBCORES = 16
NUM_WORKERS = NUM_CORES * NUM_SUBCORES
LANES = 16

B_PER_W = TOTAL // NUM_WORKERS
CHUNK = 128
N_CHUNKS = B_PER_W // CHUNK

SCALE = math.sqrt(float(D_MODEL))

_mesh = plsc.VectorSubcoreMesh(core_axis_name="c", subcore_axis_name="s")


@functools.partial(
    pl.kernel,
    mesh=_mesh,
    out_type=jax.ShapeDtypeStruct((TOTAL, D_MODEL), jnp.float32),
    scratch_types=[
        pltpu.VMEM((N_CHUNKS, CHUNK), jnp.int32),
        pltpu.VMEM((B_PER_W, D_MODEL), jnp.float32),
    ]
    + [pltpu.SemaphoreType.DMA] * (2 * N_CHUNKS),
)
def _emb_lookup(idx_hbm, table_hbm, out_hbm, idx_v, rows_v, *sems):
    g_sems = sems[:N_CHUNKS]
    w_sems = sems[N_CHUNKS:]
    wid = lax.axis_index("s") * NUM_CORES + lax.axis_index("c")
    base = wid * B_PER_W

    pltpu.sync_copy(idx_hbm.at[pl.ds(wid * N_CHUNKS, N_CHUNKS)], idx_v)

    gathers = [
        pltpu.async_copy(
            table_hbm.at[idx_v.at[j]],
            rows_v.at[pl.ds(j * CHUNK, CHUNK)],
            g_sems[j],
        )
        for j in range(N_CHUNKS)
    ]

    writes = []
    for j in range(N_CHUNKS):
        gathers[j].wait()

        def scale_rows(r, carry, j=j):
            row = j * CHUNK + r
            for c in range(D_MODEL // LANES):
                sl = pl.ds(c * LANES, LANES)
                rows_v[row, sl] = rows_v[row, sl] * SCALE
            return carry

        lax.fori_loop(0, CHUNK, scale_rows, 0, unroll=4)

        writes.append(
            pltpu.async_copy(
                rows_v.at[pl.ds(j * CHUNK, CHUNK)],
                out_hbm.at[pl.ds(base + j * CHUNK, CHUNK)],
                w_sems[j],
            )
        )
    for w in writes:
        w.wait()


def kernel(indices, table):
    idx = indices.astype(jnp.int32).reshape(NUM_WORKERS * N_CHUNKS, CHUNK)
    out = _emb_lookup(idx, table)
    return out.reshape(indices.shape + (D_MODEL,))
